# baseline (speedup 1.0000x reference)
import functools

import jax
import jax.numpy as jnp
from jax import lax
from jax.experimental import pallas as pl
from jax.experimental.pallas import tpu as pltpu
from jax.experimental.pallas import tpu_sc as plsc

SLOPE = 0.2
LANES = 16


def _proj_body(nodes_ref, ww_ref, wb_ref, ap_ref, bias_ref, ei_ref,
               out_ref, src_ref, dst_ref):
    h = lax.dot_general(nodes_ref[...], ww_ref[...], (((1,), (1,)), ((), ())),
                        preferred_element_type=jnp.float32)
    h = h + wb_ref[...]
    out_ref[...] = lax.dot_general(ap_ref[...], h, (((1,), (1,)), ((), ())),
                                   preferred_element_type=jnp.float32) + bias_ref[...]
    e = ei_ref.shape[1]
    src_ref[...] = ei_ref[0:1, :].reshape(e)
    dst_ref[...] = ei_ref[1:2, :].reshape(e)


def _node_scalars(nodes, w_w, w_b, a_w, a_b, edge_index):
    n, d_in = nodes.shape
    d_out = w_w.shape[0]
    e = edge_index.shape[1]
    a_pad = (jnp.zeros((8, d_out), jnp.float32)
             .at[0].set(a_w[0, :d_out])
             .at[1].set(a_w[0, d_out:])
             .at[2].set(1.0))
    bias = jnp.zeros((8, 1), jnp.float32).at[0, 0].set(a_b[0])
    return pl.pallas_call(
        _proj_body,
        out_shape=[
            jax.ShapeDtypeStruct((8, n), jnp.float32),
            jax.ShapeDtypeStruct((e,), jnp.int32),
            jax.ShapeDtypeStruct((e,), jnp.int32),
        ],
    )(nodes, w_w, w_b.reshape(1, d_out), a_pad, bias, edge_index)


def _edge_partials(scalars, src, dst):
    n = scalars.shape[1]
    e = src.shape[0]
    info = plsc.get_sparse_core_info()
    nc, ns = info.num_cores, info.num_subcores
    nw = nc * ns
    ept = e // nw

    def body(scal_hbm, src_hbm, dst_hbm, out_hbm,
             alpha_v, beta_v, s_v, src_v, dst_v, w_v, wsv_v, num_v, den_v):
        wid = lax.axis_index("s") * nc + lax.axis_index("c")
        pltpu.sync_copy(scal_hbm.at[0], alpha_v)
        pltpu.sync_copy(scal_hbm.at[1], beta_v)
        pltpu.sync_copy(scal_hbm.at[2], s_v)
        base = wid * ept
        pltpu.sync_copy(src_hbm.at[pl.ds(base, ept)], src_v)
        pltpu.sync_copy(dst_hbm.at[pl.ds(base, ept)], dst_v)

        zeros = jnp.zeros((LANES,), jnp.float32)

        @plsc.parallel_loop(0, n, step=LANES, unroll=4)
        def _(i):
            sl = pl.ds(i, LANES)
            num_v[sl] = zeros
            den_v[sl] = zeros

        @plsc.parallel_loop(0, ept, step=LANES, unroll=4)
        def _(g):
            sl = pl.ds(g, LANES)
            ss = src_v[sl]
            a = plsc.load_gather(alpha_v, [dst_v[sl]])
            b = plsc.load_gather(beta_v, [ss])
            sv = plsc.load_gather(s_v, [ss])
            t = a + b
            w = jnp.exp(jnp.maximum(t, t * SLOPE))
            w_v[sl] = w
            wsv_v[sl] = w * sv

        def acc_body(g, carry):
            sl = pl.ds(g * LANES, LANES)
            dd = dst_v[sl]
            plsc.addupdate_scatter(den_v, [dd], w_v[sl])
            plsc.addupdate_scatter(num_v, [dd], wsv_v[sl])
            return carry

        lax.fori_loop(0, ept // LANES, acc_body, 0)

        pltpu.sync_copy(num_v, out_hbm.at[wid])
        pltpu.sync_copy(den_v, out_hbm.at[nw + wid])

    run = functools.partial(
        pl.kernel,
        mesh=plsc.VectorSubcoreMesh(core_axis_name="c", subcore_axis_name="s"),
        compiler_params=pltpu.CompilerParams(needs_layout_passes=False),
        out_type=jax.ShapeDtypeStruct((2 * nw, n), jnp.float32),
        scratch_types=[
            pltpu.VMEM((n,), jnp.float32),
            pltpu.VMEM((n,), jnp.float32),
            pltpu.VMEM((n,), jnp.float32),
            pltpu.VMEM((ept,), jnp.int32),
            pltpu.VMEM((ept,), jnp.int32),
            pltpu.VMEM((ept,), jnp.float32),
            pltpu.VMEM((ept,), jnp.float32),
            pltpu.VMEM((n,), jnp.float32),
            pltpu.VMEM((n,), jnp.float32),
        ],
    )(body)
    return run(scalars, src, dst)


def _final_body(part_ref, scal_ref, seln_ref, seld_ref, e0_ref, e1_ref, e2_ref,
                out_ref):
    p = part_ref[...]
    sc = scal_ref[...]
    dims = (((0,), (0,)), ((), ()))
    numer = lax.dot_general(p, seln_ref[...], dims,
                            preferred_element_type=jnp.float32)
    denom = lax.dot_general(p, seld_ref[...], dims,
                            preferred_element_type=jnp.float32)
    alpha = lax.dot_general(sc, e0_ref[...], dims,
                            preferred_element_type=jnp.float32)
    beta = lax.dot_general(sc, e1_ref[...], dims,
                           preferred_element_type=jnp.float32)
    s = lax.dot_general(sc, e2_ref[...], dims,
                        preferred_element_type=jnp.float32)
    t = alpha + beta
    w = jnp.exp(jnp.maximum(t, t * SLOPE))
    r = (numer + w * s) / (denom + w)
    out_ref[...] = jnp.broadcast_to(r, out_ref.shape)


def _finalize(partials, scalars, d_out):
    two_nw, n = partials.shape
    nw = two_nw // 2
    row = jnp.arange(two_nw, dtype=jnp.int32)[:, None]
    seln = (row < nw).astype(jnp.float32)
    seld = (row >= nw).astype(jnp.float32)
    row8 = jnp.arange(8, dtype=jnp.int32)[:, None]
    e0 = (row8 == 0).astype(jnp.float32)
    e1 = (row8 == 1).astype(jnp.float32)
    e2 = (row8 == 2).astype(jnp.float32)
    return pl.pallas_call(
        _final_body,
        out_shape=jax.ShapeDtypeStruct((n, d_out), jnp.float32),
    )(partials, scalars, seln, seld, e0, e1, e2)


def kernel(nodes, edge_index, w_w, w_b, a_w, a_b):
    d_out = w_w.shape[0]
    scalars, src, dst = _node_scalars(nodes, w_w, w_b, a_w, a_b, edge_index)
    partials = _edge_partials(scalars, src, dst)
    return _finalize(partials, scalars, d_out)


# speedup vs baseline: 1.1180x; 1.1180x over previous
import functools

import jax
import jax.numpy as jnp
from jax import lax
from jax.experimental import pallas as pl
from jax.experimental.pallas import tpu as pltpu
from jax.experimental.pallas import tpu_sc as plsc

SLOPE = 0.2
LANES = 16


def _proj_body(nodes_ref, ww_ref, wb_ref, ap_ref, bias_ref, ei_ref,
               out_ref, alpha_ref, beta_ref, s_ref, src_ref, dst_ref):
    h = lax.dot_general(nodes_ref[...], ww_ref[...], (((1,), (1,)), ((), ())),
                        preferred_element_type=jnp.float32)
    h = h + wb_ref[...]
    scal = lax.dot_general(ap_ref[...], h, (((1,), (1,)), ((), ())),
                           preferred_element_type=jnp.float32) + bias_ref[...]
    out_ref[...] = scal
    n = scal.shape[1]
    alpha_ref[...] = scal[0:1, :].reshape(n)
    beta_ref[...] = scal[1:2, :].reshape(n)
    s_ref[...] = scal[2:3, :].reshape(n)
    e = ei_ref.shape[1]
    src_ref[...] = ei_ref[0:1, :].reshape(e)
    dst_ref[...] = ei_ref[1:2, :].reshape(e)


def _node_scalars(nodes, w_w, w_b, a_w, a_b, edge_index):
    n, d_in = nodes.shape
    d_out = w_w.shape[0]
    e = edge_index.shape[1]
    a_pad = (jnp.zeros((8, d_out), jnp.float32)
             .at[0].set(a_w[0, :d_out])
             .at[1].set(a_w[0, d_out:])
             .at[2].set(1.0))
    bias = jnp.zeros((8, 1), jnp.float32).at[0, 0].set(a_b[0])
    return pl.pallas_call(
        _proj_body,
        out_shape=[
            jax.ShapeDtypeStruct((8, n), jnp.float32),
            jax.ShapeDtypeStruct((n,), jnp.float32),
            jax.ShapeDtypeStruct((n,), jnp.float32),
            jax.ShapeDtypeStruct((n,), jnp.float32),
            jax.ShapeDtypeStruct((e,), jnp.int32),
            jax.ShapeDtypeStruct((e,), jnp.int32),
        ],
    )(nodes, w_w, w_b.reshape(1, d_out), a_pad, bias, edge_index)


def _edge_partials(alpha, beta, s, src, dst):
    n = alpha.shape[0]
    e = src.shape[0]
    info = plsc.get_sparse_core_info()
    nc, ns = info.num_cores, info.num_subcores
    nw = nc * ns
    ept = e // nw

    def body(alpha_hbm, beta_hbm, s_hbm, src_hbm, dst_hbm, out_hbm,
             alpha_v, beta_v, s_v, src_v, dst_v, w_v, wsv_v, num_v, den_v,
             sem):
        wid = lax.axis_index("s") * nc + lax.axis_index("c")
        base = wid * ept
        copies = [
            pltpu.async_copy(alpha_hbm, alpha_v, sem),
            pltpu.async_copy(beta_hbm, beta_v, sem),
            pltpu.async_copy(s_hbm, s_v, sem),
            pltpu.async_copy(src_hbm.at[pl.ds(base, ept)], src_v, sem),
            pltpu.async_copy(dst_hbm.at[pl.ds(base, ept)], dst_v, sem),
        ]

        zeros = jnp.zeros((LANES,), jnp.float32)

        @plsc.parallel_loop(0, n, step=LANES, unroll=4)
        def _(i):
            sl = pl.ds(i, LANES)
            num_v[sl] = zeros
            den_v[sl] = zeros

        for c in copies:
            c.wait()

        @plsc.parallel_loop(0, ept, step=LANES, unroll=4)
        def _(g):
            sl = pl.ds(g, LANES)
            ss = src_v[sl]
            a = plsc.load_gather(alpha_v, [dst_v[sl]])
            b = plsc.load_gather(beta_v, [ss])
            sv = plsc.load_gather(s_v, [ss])
            t = a + b
            w = jnp.exp(jnp.maximum(t, t * SLOPE))
            w_v[sl] = w
            wsv_v[sl] = w * sv

        def acc_body(g, carry):
            sl = pl.ds(g * LANES, LANES)
            dd = dst_v[sl]
            plsc.addupdate_scatter(den_v, [dd], w_v[sl])
            plsc.addupdate_scatter(num_v, [dd], wsv_v[sl])
            return carry

        lax.fori_loop(0, ept // LANES, acc_body, 0)

        pltpu.sync_copy(num_v, out_hbm.at[wid])
        pltpu.sync_copy(den_v, out_hbm.at[nw + wid])

    run = functools.partial(
        pl.kernel,
        mesh=plsc.VectorSubcoreMesh(core_axis_name="c", subcore_axis_name="s"),
        compiler_params=pltpu.CompilerParams(needs_layout_passes=False),
        out_type=jax.ShapeDtypeStruct((2 * nw, n), jnp.float32),
        scratch_types=[
            pltpu.VMEM((n,), jnp.float32),
            pltpu.VMEM((n,), jnp.float32),
            pltpu.VMEM((n,), jnp.float32),
            pltpu.VMEM((ept,), jnp.int32),
            pltpu.VMEM((ept,), jnp.int32),
            pltpu.VMEM((ept,), jnp.float32),
            pltpu.VMEM((ept,), jnp.float32),
            pltpu.VMEM((n,), jnp.float32),
            pltpu.VMEM((n,), jnp.float32),
            pltpu.SemaphoreType.DMA,
        ],
    )(body)
    return run(alpha, beta, s, src, dst)


def _final_body(part_ref, scal_ref, seln_ref, seld_ref, e0_ref, e1_ref, e2_ref,
                out_ref):
    p = part_ref[...]
    sc = scal_ref[...]
    dims = (((0,), (0,)), ((), ()))
    numer = lax.dot_general(p, seln_ref[...], dims,
                            preferred_element_type=jnp.float32)
    denom = lax.dot_general(p, seld_ref[...], dims,
                            preferred_element_type=jnp.float32)
    alpha = lax.dot_general(sc, e0_ref[...], dims,
                            preferred_element_type=jnp.float32)
    beta = lax.dot_general(sc, e1_ref[...], dims,
                           preferred_element_type=jnp.float32)
    s = lax.dot_general(sc, e2_ref[...], dims,
                        preferred_element_type=jnp.float32)
    t = alpha + beta
    w = jnp.exp(jnp.maximum(t, t * SLOPE))
    r = (numer + w * s) / (denom + w)
    out_ref[...] = jnp.broadcast_to(r, out_ref.shape)


def _finalize(partials, scalars, d_out):
    two_nw, n = partials.shape
    nw = two_nw // 2
    row = jnp.arange(two_nw, dtype=jnp.int32)[:, None]
    seln = (row < nw).astype(jnp.float32)
    seld = (row >= nw).astype(jnp.float32)
    row8 = jnp.arange(8, dtype=jnp.int32)[:, None]
    e0 = (row8 == 0).astype(jnp.float32)
    e1 = (row8 == 1).astype(jnp.float32)
    e2 = (row8 == 2).astype(jnp.float32)
    return pl.pallas_call(
        _final_body,
        out_shape=jax.ShapeDtypeStruct((n, d_out), jnp.float32),
    )(partials, scalars, seln, seld, e0, e1, e2)


def kernel(nodes, edge_index, w_w, w_b, a_w, a_b):
    d_out = w_w.shape[0]
    scalars, alpha, beta, s, src, dst = _node_scalars(
        nodes, w_w, w_b, a_w, a_b, edge_index)
    partials = _edge_partials(alpha, beta, s, src, dst)
    return _finalize(partials, scalars, d_out)
